# baseline (speedup 1.0000x reference)
import functools

import jax
import jax.numpy as jnp
from jax import lax
from jax.experimental import pallas as pl
from jax.experimental.pallas import tpu as pltpu
from jax.experimental.pallas import tpu_sc as plsc

G = 128
V = G * G * G
NC, NS = 2, 16
NW = NC * NS
P = 128
CF = 27


BVC = 1024


def _pack_body(feats, tab, inb, ob, sem):
    rpw = V // NW
    wid = lax.axis_index("s") * NC + lax.axis_index("c")
    off0 = wid * rpw

    @pl.loop(0, rpw // BVC)
    def _ch(t):
        off = off0 + t * BVC
        pltpu.async_copy(feats.at[pl.ds(off, BVC)], inb, sem).wait()

        @pl.loop(0, BVC, unroll=8)
        def _r(rr):
            ob[rr, pl.ds(0, 16)] = inb[rr, pl.ds(0, 16)]
            ob[rr, pl.ds(11, 16)] = inb[rr, pl.ds(11, 16)]

        pltpu.async_copy(ob, tab.at[pl.ds(off, BVC)], sem).wait()


def _sc_pack(feats):
    mesh = plsc.VectorSubcoreMesh(core_axis_name="c", subcore_axis_name="s",
                                  num_cores=NC, num_subcores=NS)
    return pl.kernel(
        _pack_body,
        out_type=jax.ShapeDtypeStruct((V, 32), jnp.float32),
        mesh=mesh,
        compiler_params=pltpu.CompilerParams(use_tc_tiling_on_sc=False),
        scratch_types=[
            pltpu.VMEM((BVC, CF), jnp.float32),
            pltpu.VMEM((BVC, 32), jnp.float32),
            pltpu.SemaphoreType.DMA,
        ],
    )(feats)


def _sc_body(n_points, feats, dens, xs, ys, zs, out,
             xb0, yb0, zb0, xb1, yb1, zb1, ib0, wb0, ib1, wb1,
             fr0, fr1, dr0, dr1, ob0, ob1, ps0, ps1, gs0, gs1, os0, os1):
    ppw = n_points // NW
    nchunks = ppw // P
    wid = lax.axis_index("s") * NC + lax.axis_index("c")
    base0 = wid * ppw
    inv64 = jnp.float32(1.0 / 64.0)
    lane = lax.iota(jnp.int32, 16)
    shift1 = jnp.minimum(lane + 1, 15)[:, None]
    is15 = lane == 15
    _gd = lax.GatherDimensionNumbers(offset_dims=(), collapsed_slice_dims=(0,),
                                     start_index_map=(0,))

    def _shift(v):
        return lax.gather(v, shift1, _gd, (1,), indices_are_sorted=True,
                          unique_indices=False,
                          mode=lax.GatherScatterMode.PROMISE_IN_BOUNDS)

    PT = ((xb0, yb0, zb0), (xb1, yb1, zb1))
    IB, WB = (ib0, ib1), (wb0, wb1)
    FR, DR, OB = (fr0, fr1), (dr0, dr1), (ob0, ob1)
    PS, GS, OS = (ps0, ps1), (gs0, gs1), (os0, os1)

    def pcopy(tc, b):
        base = base0 + jnp.minimum(tc, nchunks - 1) * P
        for src, dst in zip((xs, ys, zs), PT[b]):
            pltpu.async_copy(src.at[pl.ds(base, P)], dst, PS[b])

    def fire(tc, b):
        for src, dst in zip((xs, ys, zs), PT[b]):
            pltpu.make_async_copy(src.at[pl.ds(0, P)], dst, PS[b]).wait()
        xbuf, ybuf, zbuf = PT[b]

        @pl.loop(0, P // 16)
        def _idxw(g):
            x = xbuf[pl.ds(g * 16, 16)]
            y = ybuf[pl.ds(g * 16, 16)]
            z = zbuf[pl.ds(g * 16, 16)]
            fx = ((x * inv64 + 1.0) * jnp.float32(G) - 1.0) * jnp.float32(0.5)
            fy = ((y * inv64 + 1.0) * jnp.float32(G) - 1.0) * jnp.float32(0.5)
            fz = ((z * inv64 + 1.0) * jnp.float32(G) - 1.0) * jnp.float32(0.5)
            ix = fx.astype(jnp.int32)
            iy = fy.astype(jnp.int32)
            iz = fz.astype(jnp.int32)
            wx1 = fx - ix.astype(jnp.float32)
            wy1 = fy - iy.astype(jnp.float32)
            wz1 = fz - iz.astype(jnp.float32)
            wx0 = 1.0 - wx1
            wy0 = 1.0 - wy1
            wz0 = 1.0 - wz1
            lin = iz * (G * G) + iy * G + ix
            for k in range(8):
                dz, dy, dx = (k >> 2) & 1, (k >> 1) & 1, k & 1
                IB[b][k, pl.ds(g * 16, 16)] = lin + (dz * (G * G) + dy * G + dx)
                wk = ((wx1 if dx else wx0) * (wy1 if dy else wy0)) * (wz1 if dz else wz0)
                WB[b][k, pl.ds(g * 16, 16)] = wk

        for k in range(8):
            pltpu.async_copy(feats.at[IB[b].at[k]],
                             FR[b].at[pl.ds(k * P, P)], GS[b])
            pltpu.async_copy(dens.at[IB[b].at[k]], DR[b].at[k], GS[b])

    def combine_out(tc, b, drain_pred):
        for k in range(8):
            pltpu.make_async_copy(feats.at[IB[b].at[k]],
                                  FR[b].at[pl.ds(k * P, P)], GS[b]).wait()
            pltpu.make_async_copy(dens.at[IB[b].at[k]], DR[b].at[k],
                                  GS[b]).wait()
        base = base0 + tc * P

        def _drain():
            pltpu.make_async_copy(OB[b], out.at[pl.ds(base, P)], OS[b]).wait()
        if drain_pred is True:
            _drain()
        else:
            pl.when(drain_pred)(_drain)

        @pl.loop(0, P // 16)
        def _group(g):
            wvecs = [WB[b][k, pl.ds(g * 16, 16)] for k in range(8)]
            dacc = jnp.zeros((16,), jnp.float32)
            for k in range(8):
                dacc = dacc + wvecs[k] * jnp.abs(DR[b][k, pl.ds(g * 16, 16)])
            for j in range(16):
                p = g * 16 + j
                acc0 = jnp.zeros((16,), jnp.float32)
                acc1 = jnp.zeros((16,), jnp.float32)
                for k in range(8):
                    w = wvecs[k][j]
                    acc0 = acc0 + w * FR[b][k * P + p, pl.ds(0, 16)]
                    acc1 = acc1 + w * FR[b][k * P + p, pl.ds(11, 16)]
                hi = jnp.where(is15, dacc[j], _shift(acc1))
                OB[b][p, pl.ds(0, 16)] = acc0
                OB[b][p, pl.ds(12, 16)] = hi
            del wvecs

        pltpu.async_copy(OB[b], out.at[pl.ds(base, P)], OS[b])

    pcopy(0, 0)

    @pl.loop(0, nchunks, step=2)
    def _body(t):
        fire(t, 0)
        pcopy(t + 1, 1)

        @pl.when(t > 0)
        def _():
            combine_out(t - 1, 1, t >= 4)

        fire(t + 1, 1)

        @pl.when(t + 2 < nchunks)
        def _():
            pcopy(t + 2, 0)

        combine_out(t, 0, t >= 2)

    combine_out(nchunks - 1, 1, True)
    pltpu.make_async_copy(ob0, out.at[pl.ds(0, P)], os0).wait()
    pltpu.make_async_copy(ob1, out.at[pl.ds(0, P)], os1).wait()


def _sc_sample(feats, dens, xs, ys, zs):
    n = xs.shape[0]
    mesh = plsc.VectorSubcoreMesh(core_axis_name="c", subcore_axis_name="s",
                                  num_cores=NC, num_subcores=NS)
    return pl.kernel(
        functools.partial(_sc_body, n),
        out_type=jax.ShapeDtypeStruct((n, 28), jnp.float32),
        mesh=mesh,
        compiler_params=pltpu.CompilerParams(use_tc_tiling_on_sc=False),
        scratch_types=[
            pltpu.VMEM((P,), jnp.float32), pltpu.VMEM((P,), jnp.float32),
            pltpu.VMEM((P,), jnp.float32), pltpu.VMEM((P,), jnp.float32),
            pltpu.VMEM((P,), jnp.float32), pltpu.VMEM((P,), jnp.float32),
            pltpu.VMEM((8, P), jnp.int32), pltpu.VMEM((8, P), jnp.float32),
            pltpu.VMEM((8, P), jnp.int32), pltpu.VMEM((8, P), jnp.float32),
            pltpu.VMEM((8 * P, 32), jnp.float32),
            pltpu.VMEM((8 * P, 32), jnp.float32),
            pltpu.VMEM((8, P), jnp.float32),
            pltpu.VMEM((8, P), jnp.float32),
            pltpu.VMEM((P, 28), jnp.float32),
            pltpu.VMEM((P, 28), jnp.float32),
            pltpu.SemaphoreType.DMA, pltpu.SemaphoreType.DMA,
            pltpu.SemaphoreType.DMA, pltpu.SemaphoreType.DMA,
            pltpu.SemaphoreType.DMA, pltpu.SemaphoreType.DMA,
        ],
    )(feats, dens, xs, ys, zs)


def kernel(points, densities, features):
    tab = _sc_pack(features.reshape(V, CF))
    dens = densities.reshape(V)
    xs, ys, zs = points[:, 0], points[:, 1], points[:, 2]
    return _sc_sample(tab, dens, xs, ys, zs)


# speedup vs baseline: 1.3210x; 1.3210x over previous
import functools

import jax
import jax.numpy as jnp
from jax import lax
from jax.experimental import pallas as pl
from jax.experimental.pallas import tpu as pltpu
from jax.experimental.pallas import tpu_sc as plsc

G = 128
V = G * G * G
NC, NS = 2, 16
NW = NC * NS
P = 128
CF = 27


def _sc_body(n_points, feats, xs, ys, zs, out,
             xb0, yb0, zb0, xb1, yb1, zb1, ib0, wb0, ib1, wb1,
             fr0, fr1, ob0, ob1, ps0, ps1, gs0, gs1, os0, os1):
    ppw = n_points // NW
    nchunks = ppw // P
    wid = lax.axis_index("s") * NC + lax.axis_index("c")
    base0 = wid * ppw
    inv64 = jnp.float32(1.0 / 64.0)

    PT = ((xb0, yb0, zb0), (xb1, yb1, zb1))
    IB, WB = (ib0, ib1), (wb0, wb1)
    FR, OB = (fr0, fr1), (ob0, ob1)
    PS, GS, OS = (ps0, ps1), (gs0, gs1), (os0, os1)

    def pcopy(tc, b):
        base = base0 + jnp.minimum(tc, nchunks - 1) * P
        for src, dst in zip((xs, ys, zs), PT[b]):
            pltpu.async_copy(src.at[pl.ds(base, P)], dst, PS[b])

    def fire(tc, b):
        for src, dst in zip((xs, ys, zs), PT[b]):
            pltpu.make_async_copy(src.at[pl.ds(0, P)], dst, PS[b]).wait()
        xbuf, ybuf, zbuf = PT[b]

        @pl.loop(0, P // 16)
        def _idxw(g):
            x = xbuf[pl.ds(g * 16, 16)]
            y = ybuf[pl.ds(g * 16, 16)]
            z = zbuf[pl.ds(g * 16, 16)]
            fx = ((x * inv64 + 1.0) * jnp.float32(G) - 1.0) * jnp.float32(0.5)
            fy = ((y * inv64 + 1.0) * jnp.float32(G) - 1.0) * jnp.float32(0.5)
            fz = ((z * inv64 + 1.0) * jnp.float32(G) - 1.0) * jnp.float32(0.5)
            ix = fx.astype(jnp.int32)
            iy = fy.astype(jnp.int32)
            iz = fz.astype(jnp.int32)
            wx1 = fx - ix.astype(jnp.float32)
            wy1 = fy - iy.astype(jnp.float32)
            wz1 = fz - iz.astype(jnp.float32)
            wx0 = 1.0 - wx1
            wy0 = 1.0 - wy1
            wz0 = 1.0 - wz1
            lin = iz * (G * G) + iy * G + ix
            for k in range(8):
                dz, dy, dx = (k >> 2) & 1, (k >> 1) & 1, k & 1
                IB[b][k, pl.ds(g * 16, 16)] = lin + (dz * (G * G) + dy * G + dx)
                wk = ((wx1 if dx else wx0) * (wy1 if dy else wy0)) * (wz1 if dz else wz0)
                WB[b][k, pl.ds(g * 16, 16)] = wk

        for k in range(8):
            pltpu.async_copy(feats.at[IB[b].at[k]],
                             FR[b].at[pl.ds(k * P, P)], GS[b])

    def combine_out(tc, b, drain_pred):
        for k in range(8):
            pltpu.make_async_copy(feats.at[IB[b].at[k]],
                                  FR[b].at[pl.ds(k * P, P)], GS[b]).wait()
        base = base0 + tc * P

        def _drain():
            pltpu.make_async_copy(OB[b], out.at[pl.ds(base, P)], OS[b]).wait()
        if drain_pred is True:
            _drain()
        else:
            pl.when(drain_pred)(_drain)

        @pl.loop(0, P // 16)
        def _group(g):
            wvecs = [WB[b][k, pl.ds(g * 16, 16)] for k in range(8)]
            for j in range(16):
                p = g * 16 + j
                acc0 = jnp.zeros((16,), jnp.float32)
                acc1 = jnp.zeros((16,), jnp.float32)
                for k in range(8):
                    w = wvecs[k][j]
                    acc0 = acc0 + w * FR[b][k * P + p, pl.ds(0, 16)]
                    acc1 = acc1 + w * FR[b][k * P + p, pl.ds(12, 16)]
                OB[b][p, pl.ds(0, 16)] = acc0
                OB[b][p, pl.ds(12, 16)] = acc1

        pltpu.async_copy(OB[b], out.at[pl.ds(base, P)], OS[b])

    pcopy(0, 0)

    @pl.loop(0, nchunks, step=2)
    def _body(t):
        fire(t, 0)
        pcopy(t + 1, 1)

        @pl.when(t > 0)
        def _():
            combine_out(t - 1, 1, t >= 4)

        fire(t + 1, 1)

        @pl.when(t + 2 < nchunks)
        def _():
            pcopy(t + 2, 0)

        combine_out(t, 0, t >= 2)

    combine_out(nchunks - 1, 1, True)
    pltpu.make_async_copy(ob0, out.at[pl.ds(0, P)], os0).wait()
    pltpu.make_async_copy(ob1, out.at[pl.ds(0, P)], os1).wait()


def _sc_sample(feats, xs, ys, zs):
    n = xs.shape[0]
    mesh = plsc.VectorSubcoreMesh(core_axis_name="c", subcore_axis_name="s",
                                  num_cores=NC, num_subcores=NS)
    return pl.kernel(
        functools.partial(_sc_body, n),
        out_type=jax.ShapeDtypeStruct((n, 28), jnp.float32),
        mesh=mesh,
        compiler_params=pltpu.CompilerParams(use_tc_tiling_on_sc=False),
        scratch_types=[
            pltpu.VMEM((P,), jnp.float32), pltpu.VMEM((P,), jnp.float32),
            pltpu.VMEM((P,), jnp.float32), pltpu.VMEM((P,), jnp.float32),
            pltpu.VMEM((P,), jnp.float32), pltpu.VMEM((P,), jnp.float32),
            pltpu.VMEM((8, P), jnp.int32), pltpu.VMEM((8, P), jnp.float32),
            pltpu.VMEM((8, P), jnp.int32), pltpu.VMEM((8, P), jnp.float32),
            pltpu.VMEM((8 * P, 32), jnp.float32),
            pltpu.VMEM((8 * P, 32), jnp.float32),
            pltpu.VMEM((P, 28), jnp.float32),
            pltpu.VMEM((P, 28), jnp.float32),
            pltpu.SemaphoreType.DMA, pltpu.SemaphoreType.DMA,
            pltpu.SemaphoreType.DMA, pltpu.SemaphoreType.DMA,
            pltpu.SemaphoreType.DMA, pltpu.SemaphoreType.DMA,
        ],
    )(feats, xs, ys, zs)


def kernel(points, densities, features):
    table = jnp.concatenate(
        [features.reshape(V, CF), jnp.abs(densities).reshape(V, 1),
         jnp.zeros((V, 32 - CF - 1), jnp.float32)], axis=1)
    xs, ys, zs = points[:, 0], points[:, 1], points[:, 2]
    return _sc_sample(table, xs, ys, zs)
